# baseline (speedup 1.0000x reference)
import functools

import jax
import jax.numpy as jnp
from jax import lax
from jax.experimental import pallas as pl
from jax.experimental.pallas import tpu as pltpu
from jax.experimental.pallas import tpu_sc as plsc

VOCAB = 100000
D_MODEL = 128

_info = plsc.get_sparse_core_info()
_NC, _NS = _info.num_cores, _info.num_subcores
_NW = _NC * _NS

_CHUNK = 128

_NBUF = 2


@functools.partial(jax.jit, static_argnames=("b_per_w",))
def _gather_sc(x_flat, table, *, b_per_w):
    n_chunks = b_per_w // _CHUNK
    B = _NW * b_per_w
    mesh = plsc.VectorSubcoreMesh(core_axis_name="c", subcore_axis_name="s")

    @functools.partial(
        pl.kernel,
        mesh=mesh,
        out_type=jax.ShapeDtypeStruct((B // _CHUNK, _CHUNK, D_MODEL), jnp.float32),
        scratch_types=[
            pltpu.VMEM((n_chunks, _CHUNK), jnp.int32),
            pltpu.VMEM((4, _CHUNK, D_MODEL), jnp.float32),
            pltpu.SemaphoreType.DMA((4,)),
            pltpu.SemaphoreType.DMA((2,)),
            pltpu.SemaphoreType.DMA,
        ],
    )
    def k(x_hbm, table_hbm, out_hbm, idx_v, rows_v, gsem, osem, isem):
        wid = lax.axis_index("s") * _NC + lax.axis_index("c")
        cbase = wid * n_chunks

        staged = pltpu.make_async_copy(x_hbm.at[wid], idx_v.at[...], isem)
        staged.start()
        staged.wait()

        def gather(j, slot):
            return pltpu.make_async_copy(
                table_hbm.at[idx_v.at[j]],
                rows_v.at[slot],
                gsem.at[slot],
            )

        def store_pair(p, sp):
            return pltpu.make_async_copy(
                rows_v.at[pl.ds(2 * sp, 2)],
                out_hbm.at[pl.ds(cbase + 2 * p, 2)],
                osem.at[sp],
            )

        gather(0, 0).start()
        gather(1, 1).start()
        n_pairs = n_chunks // 2

        def body(p, _):
            sp = lax.rem(p, 2)
            osp = lax.rem(p + 1, 2)

            @pl.when(p > 0)
            def _():
                store_pair(p - 1, osp).wait()

            @pl.when(p + 1 < n_pairs)
            def _():
                gather(2 * p + 2, 2 * osp).start()
                gather(2 * p + 3, 2 * osp + 1).start()

            gather(2 * p, 2 * sp).wait()
            gather(2 * p + 1, 2 * sp + 1).wait()
            store_pair(p, sp).start()
            return 0

        lax.fori_loop(0, n_pairs, body, 0)
        store_pair(n_pairs - 1, lax.rem(n_pairs - 1, 2)).wait()

    return k(x_flat, table)


def kernel(x, table):
    B_total = x.shape[0] * x.shape[1]
    x_flat = jnp.reshape(x.astype(jnp.int32), (_NW, B_total // (_NW * _CHUNK), _CHUNK))
    b_per_w = B_total // _NW
    out = _gather_sc(x_flat, table, b_per_w=b_per_w)
    return jnp.reshape(out, (x.shape[0], x.shape[1], D_MODEL))
